# Initial kernel scaffold; baseline (speedup 1.0000x reference)
import jax
import jax.numpy as jnp
from jax.experimental import pallas as pl


def kernel(x, edge_index, batch, params):
    raise NotImplementedError("write your pallas kernel here")


# speedup vs baseline: 4.1110x; 4.1110x over previous
import functools

import jax
import jax.numpy as jnp
from jax import lax
from jax.experimental import pallas as pl
from jax.experimental.pallas import tpu as pltpu
from jax.experimental.pallas import tpu_sc as plsc

_N = 10000
_E = 320000
_D = 128
_O = 64
_G = 128

_NC = 2
_NS = 16
_NW = _NC * _NS
_CHUNK = 128
_RPW = -(-_E // (_NW * _CHUNK))
_E_PAD = _NW * _RPW * _CHUNK
_N_ACC = 10240
_RPS = _N_ACC // _NS

_BR = 1000
_NB = _N // _BR


def _make_agg():
    mesh = plsc.VectorSubcoreMesh(
        core_axis_name="c", subcore_axis_name="s",
        num_cores=_NC, num_subcores=_NS)

    @functools.partial(
        pl.kernel,
        out_type=jax.ShapeDtypeStruct((_NC, _N_ACC, _D), jnp.float32),
        mesh=mesh,
        scratch_types=[
            pltpu.VMEM_SHARED((_N_ACC, _D), jnp.float32),
            pltpu.VMEM((_RPW, _CHUNK), jnp.int32),
            pltpu.VMEM((_RPW, _CHUNK), jnp.int32),
            pltpu.VMEM((_CHUNK, _D), jnp.float32),
            pltpu.SemaphoreType.DMA,
        ],
    )
    def agg(h_hbm, srcp_hbm, dstp_hbm, zeros_hbm, out_hbm,
            accum, src_v, dst_v, rows_v, sem):
        c = lax.axis_index("c")
        s = lax.axis_index("s")
        wid = c * _NS + s
        pltpu.sync_copy(zeros_hbm.at[pl.ds(s * _RPS, _RPS)],
                        accum.at[pl.ds(s * _RPS, _RPS)])
        pltpu.sync_copy(srcp_hbm.at[wid], src_v)
        pltpu.sync_copy(dstp_hbm.at[wid], dst_v)
        plsc.subcore_barrier()

        def body(j, carry):
            pltpu.async_copy(h_hbm.at[src_v.at[j]], rows_v, sem).wait()
            pltpu.sync_copy(rows_v, accum.at[dst_v.at[j]], add=True)
            return carry

        lax.fori_loop(0, _RPW, body, 0)
        plsc.subcore_barrier()
        pltpu.sync_copy(accum.at[pl.ds(s * _RPS, _RPS)],
                        out_hbm.at[c].at[pl.ds(s * _RPS, _RPS)])

    return agg


_agg = _make_agg()


def _mlp_body(eps_ref, h_ref, p_ref, w0_ref, b0_ref, w1_ref, b1_ref, out_ref):
    z = eps_ref[0, 0] * h_ref[...] + p_ref[0] + p_ref[1]
    t = jnp.dot(z, w0_ref[...], preferred_element_type=jnp.float32)
    t = jnp.maximum(t + b0_ref[...], 0.0)
    u = jnp.dot(t, w1_ref[...], preferred_element_type=jnp.float32)
    out_ref[...] = jnp.maximum(u + b1_ref[...], 0.0)


def _mlp(epsp, h, p, w0, b0, w1, b1):
    return pl.pallas_call(
        _mlp_body,
        grid=(_NB,),
        in_specs=[
            pl.BlockSpec((1, 1), lambda i: (0, 0), memory_space=pltpu.SMEM),
            pl.BlockSpec((_BR, _D), lambda i: (i, 0)),
            pl.BlockSpec((_NC, _BR, _D), lambda i: (0, i, 0)),
            pl.BlockSpec((_D, _D), lambda i: (0, 0)),
            pl.BlockSpec((1, _D), lambda i: (0, 0)),
            pl.BlockSpec((_D, _D), lambda i: (0, 0)),
            pl.BlockSpec((1, _D), lambda i: (0, 0)),
        ],
        out_specs=pl.BlockSpec((_BR, _D), lambda i: (i, 0)),
        out_shape=jax.ShapeDtypeStruct((_N, _D), jnp.float32),
    )(epsp, h, p, w0, b0, w1, b1)


def _pool_body(batch_ref, x_ref, h1_ref, h2_ref, wp0_ref, wp1_ref, wp2_ref,
               bsum_ref, out_ref):
    i = pl.program_id(0)
    s = jnp.dot(x_ref[...], wp0_ref[...], preferred_element_type=jnp.float32)
    s += jnp.dot(h1_ref[...], wp1_ref[...], preferred_element_type=jnp.float32)
    s += jnp.dot(h2_ref[...], wp2_ref[...], preferred_element_type=jnp.float32)
    bid = batch_ref[0, 0, :]
    onehot = (bid[:, None] == lax.broadcasted_iota(jnp.int32, (_BR, _G), 1))
    onehot = onehot.astype(jnp.float32)
    contrib = lax.dot_general(onehot, s, (((0,), (0,)), ((), ())),
                              preferred_element_type=jnp.float32)

    @pl.when(i == 0)
    def _():
        out_ref[...] = jnp.broadcast_to(bsum_ref[...], (_G, _O))

    out_ref[...] += contrib


def _pool(batch3, x, h1, h2, wp0, wp1, wp2, bsum):
    return pl.pallas_call(
        _pool_body,
        grid=(_NB,),
        in_specs=[
            pl.BlockSpec((1, 1, _BR), lambda i: (i, 0, 0)),
            pl.BlockSpec((_BR, _D), lambda i: (i, 0)),
            pl.BlockSpec((_BR, _D), lambda i: (i, 0)),
            pl.BlockSpec((_BR, _D), lambda i: (i, 0)),
            pl.BlockSpec((_D, _O), lambda i: (0, 0)),
            pl.BlockSpec((_D, _O), lambda i: (0, 0)),
            pl.BlockSpec((_D, _O), lambda i: (0, 0)),
            pl.BlockSpec((1, _O), lambda i: (0, 0)),
        ],
        out_specs=pl.BlockSpec((_G, _O), lambda i: (0, 0)),
        out_shape=jax.ShapeDtypeStruct((_G, _O), jnp.float32),
    )(batch3, x, h1, h2, wp0, wp1, wp2, bsum)


def kernel(x, edge_index, batch, params):
    src = edge_index[0]
    dst = edge_index[1]
    npad = _E_PAD - _E
    pad_src = jnp.zeros((npad,), jnp.int32)
    pad_dst = _N + (jnp.arange(npad, dtype=jnp.int32) % (_N_ACC - _N))
    srcp = jnp.concatenate([src, pad_src]).reshape(_NW, _RPW, _CHUNK)
    dstp = jnp.concatenate([dst, pad_dst]).reshape(_NW, _RPW, _CHUNK)
    zeros = jnp.zeros((_N_ACC, _D), jnp.float32)
    batch3 = batch.reshape(_NB, 1, _BR)

    cbn = 1.0 / jnp.sqrt(1.0 + 1e-5)
    folded = []
    for l in range(2):
        g0 = params[f"mlp_g{l}"] * cbn
        w0 = params[f"W0_{l}"] * g0[None, :]
        b0 = (params[f"b0_{l}"] * g0 + params[f"mlp_b{l}"]).reshape(1, _D)
        g1 = params[f"g{l}"] * cbn
        w1 = params[f"W1_{l}"] * g1[None, :]
        b1 = (params[f"b1_{l}"] * g1 + params[f"b{l}"]).reshape(1, _D)
        epsp = (1.0 + params[f"eps{l}"]).reshape(1, 1)
        folded.append((epsp, w0, b0, w1, b1))

    h = x
    hidden = [x]
    for l in range(2):
        p = _agg(h, srcp, dstp, zeros)
        epsp, w0, b0, w1, b1 = folded[l]
        h = _mlp(epsp, h, p, w0, b0, w1, b1)
        hidden.append(h)

    bsum = (params["bp0"] + params["bp1"] + params["bp2"]).reshape(1, _O)
    return _pool(batch3, hidden[0], hidden[1], hidden[2],
                 params["Wp0"], params["Wp1"], params["Wp2"], bsum)
